# Initial kernel scaffold; baseline (speedup 1.0000x reference)
import functools

import jax
import jax.numpy as jnp
from jax import lax
from jax.experimental import pallas as pl
from jax.experimental.pallas import tpu as pltpu
from jax.experimental.pallas import tpu_sc as plsc

VOCAB = 1000
D = 1000
B_TOTAL = 1024 * 50
NC = 2
NS = 16
NW = NC * NS
B_PER_W = B_TOTAL // NW
C = 32
N_CHUNKS = B_PER_W // C


def _make_gather():
    mesh = plsc.VectorSubcoreMesh(core_axis_name="c", subcore_axis_name="s")

    @functools.partial(
        pl.kernel,
        mesh=mesh,
        out_type=jax.ShapeDtypeStruct((B_TOTAL, D), jnp.float32),
        scratch_types=[
            pltpu.VMEM((N_CHUNKS, C), jnp.int32),
            pltpu.VMEM((C, D), jnp.float32),
            pltpu.VMEM((C, D), jnp.float32),
            pltpu.SemaphoreType.DMA,
            pltpu.SemaphoreType.DMA,
            pltpu.SemaphoreType.DMA,
            pltpu.SemaphoreType.DMA,
        ],
    )
    def k(idx_hbm, table_hbm, out_hbm, idx_v, buf0, buf1, g0, g1, o0, o1):
        wid = lax.axis_index("s") * NC + lax.axis_index("c")
        base = wid * B_PER_W
        pltpu.sync_copy(idx_hbm.at[wid], idx_v)

        bufs = (buf0, buf1)
        gsems = (g0, g1)
        osems = (o0, o1)

        pltpu.async_copy(table_hbm.at[idx_v.at[0]], buf0, g0)
        pltpu.async_copy(table_hbm.at[idx_v.at[1]], buf1, g1)

        @pl.loop(0, N_CHUNKS, step=2)
        def _(c):
            for b in range(2):
                cc = c + b
                pltpu.make_async_copy(
                    table_hbm.at[idx_v.at[cc]], bufs[b], gsems[b]
                ).wait()
                pltpu.async_copy(
                    bufs[b], out_hbm.at[pl.ds(base + cc * C, C)], osems[b]
                )
                pltpu.make_async_copy(
                    bufs[b], out_hbm.at[pl.ds(base + cc * C, C)], osems[b]
                ).wait()
                @pl.when(cc + 2 < N_CHUNKS)
                def _():
                    pltpu.async_copy(
                        table_hbm.at[idx_v.at[cc + 2]], bufs[b], gsems[b]
                    )

    return k


_gather = _make_gather()


def kernel(index, table):
    idx = index.reshape(NW, N_CHUNKS, C)
    out = _gather(idx, table)
    return out.reshape(index.shape[0], index.shape[1], VOCAB)


# speedup vs baseline: 1.0364x; 1.0364x over previous
import functools

import jax
import jax.numpy as jnp
from jax import lax
from jax.experimental import pallas as pl
from jax.experimental.pallas import tpu as pltpu
from jax.experimental.pallas import tpu_sc as plsc

VOCAB = 1000
D = 1000
B_TOTAL = 1024 * 50
NC = 2
NS = 16
NW = NC * NS
B_PER_W = B_TOTAL // NW
C = 32
N_CHUNKS = B_PER_W // C


def _make_gather():
    mesh = plsc.VectorSubcoreMesh(core_axis_name="c", subcore_axis_name="s")

    @functools.partial(
        pl.kernel,
        mesh=mesh,
        compiler_params=pltpu.CompilerParams(use_tc_tiling_on_sc=False),
        out_type=jax.ShapeDtypeStruct((B_TOTAL, D), jnp.float32),
        scratch_types=[
            pltpu.VMEM((N_CHUNKS, C), jnp.int32),
            pltpu.VMEM((C, D), jnp.float32),
            pltpu.VMEM((C, D), jnp.float32),
            pltpu.SemaphoreType.DMA,
            pltpu.SemaphoreType.DMA,
            pltpu.SemaphoreType.DMA,
            pltpu.SemaphoreType.DMA,
        ],
    )
    def k(idx_hbm, table_hbm, out_hbm, idx_v, buf0, buf1, g0, g1, o0, o1):
        wid = lax.axis_index("s") * NC + lax.axis_index("c")
        base = wid * B_PER_W
        pltpu.sync_copy(idx_hbm.at[wid], idx_v)

        bufs = (buf0, buf1)
        gsems = (g0, g1)
        osems = (o0, o1)

        pltpu.async_copy(table_hbm.at[idx_v.at[0]], buf0, g0)
        pltpu.async_copy(table_hbm.at[idx_v.at[1]], buf1, g1)

        @pl.loop(0, N_CHUNKS, step=2)
        def _(c):
            for b in range(2):
                cc = c + b
                pltpu.make_async_copy(
                    table_hbm.at[idx_v.at[cc]], bufs[b], gsems[b]
                ).wait()
                pltpu.async_copy(
                    bufs[b], out_hbm.at[pl.ds(base + cc * C, C)], osems[b]
                )
                pltpu.make_async_copy(
                    bufs[b], out_hbm.at[pl.ds(base + cc * C, C)], osems[b]
                ).wait()
                @pl.when(cc + 2 < N_CHUNKS)
                def _():
                    pltpu.async_copy(
                        table_hbm.at[idx_v.at[cc + 2]], bufs[b], gsems[b]
                    )

    return k


_gather = _make_gather()


def kernel(index, table):
    idx = index.reshape(NW, N_CHUNKS, C)
    out = _gather(idx, table)
    return out.reshape(index.shape[0], index.shape[1], VOCAB)


# speedup vs baseline: 1.4189x; 1.3692x over previous
import functools

import jax
import jax.numpy as jnp
from jax import lax
from jax.experimental import pallas as pl
from jax.experimental.pallas import tpu as pltpu
from jax.experimental.pallas import tpu_sc as plsc

VOCAB = 1000
D = 1000
DP = 1024
DM = 896
DT = D - DM
B_TOTAL = 1024 * 50
NC = 2
NS = 16
NW = NC * NS
B_PER_W = B_TOTAL // NW
C = 32
N_CHUNKS = B_PER_W // C

_TAIL_OFFS = (0, 16, 32, 48, 64, 80, 88)


def _make_gather():
    mesh = plsc.VectorSubcoreMesh(core_axis_name="c", subcore_axis_name="s")

    @functools.partial(
        pl.kernel,
        mesh=mesh,
        out_type=jax.ShapeDtypeStruct((B_TOTAL, D), jnp.float32),
        scratch_types=[
            pltpu.VMEM((N_CHUNKS, C), jnp.int32),
            pltpu.VMEM((C, DP), jnp.float32),
            pltpu.VMEM((C, DP), jnp.float32),
            pltpu.VMEM((C, DT), jnp.float32),
            pltpu.VMEM((C, DT), jnp.float32),
            pltpu.SemaphoreType.DMA,
            pltpu.SemaphoreType.DMA,
            pltpu.SemaphoreType.DMA,
            pltpu.SemaphoreType.DMA,
            pltpu.SemaphoreType.DMA,
            pltpu.SemaphoreType.DMA,
        ],
    )
    def k(idx_hbm, table_hbm, out_hbm, idx_v,
          bufg0, bufg1, buft0, buft1, g0, g1, m0, m1, t0, t1):
        wid = lax.axis_index("s") * NC + lax.axis_index("c")
        base = wid * B_PER_W
        pltpu.sync_copy(idx_hbm.at[wid], idx_v)

        bufgs = (bufg0, bufg1)
        bufts = (buft0, buft1)
        gsems = (g0, g1)
        msems = (m0, m1)
        tsems = (t0, t1)

        pltpu.async_copy(table_hbm.at[idx_v.at[0]], bufg0, g0)
        pltpu.async_copy(table_hbm.at[idx_v.at[1]], bufg1, g1)

        @pl.loop(0, N_CHUNKS, step=2)
        def _(c):
            for b in range(2):
                cc = c + b
                bufg, buft = bufgs[b], bufts[b]
                pltpu.make_async_copy(
                    table_hbm.at[idx_v.at[cc]], bufg, gsems[b]
                ).wait()
                for r in range(C):
                    for j in _TAIL_OFFS:
                        buft[r, pl.ds(j, 16)] = bufg[r, pl.ds(DM + j, 16)]
                srcm = bufg.at[:, pl.ds(0, DM)]
                dstm = out_hbm.at[pl.ds(base + cc * C, C), pl.ds(0, DM)]
                pltpu.async_copy(srcm, dstm, msems[b])
                dstt = out_hbm.at[pl.ds(base + cc * C, C), pl.ds(DM, DT)]
                pltpu.async_copy(buft, dstt, tsems[b])
                pltpu.make_async_copy(srcm, dstm, msems[b]).wait()
                pltpu.make_async_copy(buft, dstt, tsems[b]).wait()

                @pl.when(cc + 2 < N_CHUNKS)
                def _():
                    pltpu.async_copy(
                        table_hbm.at[idx_v.at[cc + 2]], bufg, gsems[b]
                    )

    return k


_gather = _make_gather()


def kernel(index, table):
    idx = index.reshape(NW, N_CHUNKS, C)
    table_p = jnp.pad(table, ((0, 0), (0, DP - D)))
    out = _gather(idx, table_p)
    return out.reshape(index.shape[0], index.shape[1], VOCAB)


# speedup vs baseline: 1.7767x; 1.2522x over previous
import functools

import jax
import jax.numpy as jnp
from jax import lax
from jax.experimental import pallas as pl
from jax.experimental.pallas import tpu as pltpu
from jax.experimental.pallas import tpu_sc as plsc

VOCAB = 1000
D = 1000
DP = 1024
DM = 896
DT = D - DM
B = 1024
T = 50
TA = 48
TD = T - TA
TP = 64
NC = 2
NS = 16
NW = NC * NS
N_CHUNKS = B // NW

_TAIL_OFFS = (0, 16, 32, 48, 64, 80, 88)


def _make_gather():
    mesh = plsc.VectorSubcoreMesh(core_axis_name="c", subcore_axis_name="s")

    @functools.partial(
        pl.kernel,
        mesh=mesh,
        out_type=jax.ShapeDtypeStruct((B, T, D), jnp.float32),
        scratch_types=[
            pltpu.VMEM((N_CHUNKS, TP), jnp.int32),
            pltpu.VMEM((TA, DP), jnp.float32),
            pltpu.VMEM((TA, DP), jnp.float32),
            pltpu.VMEM((TD, DP), jnp.float32),
            pltpu.VMEM((TD, DP), jnp.float32),
            pltpu.VMEM((TA, DT), jnp.float32),
            pltpu.VMEM((TD, DT), jnp.float32),
            pltpu.SemaphoreType.DMA,
            pltpu.SemaphoreType.DMA,
            pltpu.SemaphoreType.DMA,
        ],
    )
    def k(idx_hbm, table_hbm, out_hbm, idx_v,
          bufa0, bufa1, bufd0, bufd1, bufta, buftd, g0, g1, osem):
        wid = lax.axis_index("s") * NC + lax.axis_index("c")
        base = wid * N_CHUNKS
        pltpu.sync_copy(idx_hbm.at[wid], idx_v)

        bufas = (bufa0, bufa1)
        bufds = (bufd0, bufd1)
        gsems = (g0, g1)

        def start_gathers(cc, b):
            pltpu.async_copy(
                table_hbm.at[idx_v.at[cc, pl.ds(0, TA)]], bufas[b], gsems[b])
            pltpu.async_copy(
                table_hbm.at[idx_v.at[cc, pl.ds(56, TD)]], bufds[b], gsems[b])

        def wait_gathers(cc, b):
            pltpu.make_async_copy(
                table_hbm.at[idx_v.at[cc, pl.ds(0, TA)]], bufas[b],
                gsems[b]).wait()
            pltpu.make_async_copy(
                table_hbm.at[idx_v.at[cc, pl.ds(56, TD)]], bufds[b],
                gsems[b]).wait()

        start_gathers(0, 0)
        start_gathers(1, 1)

        @pl.loop(0, N_CHUNKS, step=2)
        def _(c):
            for b in range(2):
                cc = c + b
                bufa, bufd = bufas[b], bufds[b]
                wait_gathers(cc, b)
                for r in range(TA):
                    for j in _TAIL_OFFS:
                        bufta[r, pl.ds(j, 16)] = bufa[r, pl.ds(DM + j, 16)]
                for r in range(TD):
                    for j in _TAIL_OFFS:
                        buftd[r, pl.ds(j, 16)] = bufd[r, pl.ds(DM + j, 16)]
                srcs = (
                    bufa.at[:, pl.ds(0, DM)],
                    bufd.at[:, pl.ds(0, DM)],
                    bufta,
                    buftd,
                )
                dsts = (
                    out_hbm.at[base + cc, pl.ds(0, TA), pl.ds(0, DM)],
                    out_hbm.at[base + cc, pl.ds(TA, TD), pl.ds(0, DM)],
                    out_hbm.at[base + cc, pl.ds(0, TA), pl.ds(DM, DT)],
                    out_hbm.at[base + cc, pl.ds(TA, TD), pl.ds(DM, DT)],
                )
                for s, d in zip(srcs, dsts):
                    pltpu.async_copy(s, d, osem)
                for s, d in zip(srcs, dsts):
                    pltpu.make_async_copy(s, d, osem).wait()

                @pl.when(cc + 2 < N_CHUNKS)
                def _():
                    start_gathers(cc + 2, b)

    return k


_gather = _make_gather()


def kernel(index, table):
    z6 = jnp.zeros((B, 6), jnp.int32)
    idx = jnp.concatenate([index, z6, index[:, TA:T], z6], axis=1)
    idx = idx.reshape(NW, N_CHUNKS, TP)
    table_p = jnp.pad(table, ((0, 0), (0, DP - D)))
    return _gather(idx, table_p)
